# Initial kernel scaffold; baseline (speedup 1.0000x reference)
import jax
import jax.numpy as jnp
from jax.experimental import pallas as pl


def kernel(idseq, length_list, emb_table):
    raise NotImplementedError("write your pallas kernel here")


# speedup vs baseline: 21.1686x; 21.1686x over previous
import functools

import jax
import jax.numpy as jnp
from jax import lax
from jax.experimental import pallas as pl
from jax.experimental.pallas import tpu as pltpu
from jax.experimental.pallas import tpu_sc as plsc

_NC = 2
_NS = 16
_NW = _NC * _NS
_L = 16


def _rowsum_body(emb_ref, out_ref):
    x = emb_ref[...]
    out_ref[...] = jax.nn.sigmoid(jnp.sum(x, axis=1, keepdims=True))


def _rowsum_sigmoid(emb_table, blk):
    v, d = emb_table.shape
    grid = v // blk
    return pl.pallas_call(
        _rowsum_body,
        grid=(grid,),
        in_specs=[pl.BlockSpec((blk, d), lambda i: (i, 0))],
        out_specs=pl.BlockSpec((blk, 1), lambda i: (i, 0)),
        out_shape=jax.ShapeDtypeStruct((v, 1), jnp.float32),
    )(emb_table)


def _make_sc_gather(v, total, chunk):
    per_w = total // _NW
    n_chunks = per_w // chunk
    mesh = plsc.VectorSubcoreMesh(core_axis_name="c", subcore_axis_name="s")

    @functools.partial(
        pl.kernel,
        mesh=mesh,
        out_type=jax.ShapeDtypeStruct((total,), jnp.float32),
        scratch_types=[
            pltpu.VMEM((v,), jnp.float32),
            pltpu.VMEM((chunk,), jnp.int32),
            pltpu.VMEM((chunk,), jnp.float32),
        ],
        compiler_params=pltpu.CompilerParams(needs_layout_passes=False),
    )
    def gather_kernel(s_hbm, idx_hbm, out_hbm, s_v, idx_v, out_v):
        wid = lax.axis_index("s") * _NC + lax.axis_index("c")
        base = wid * per_w
        pltpu.sync_copy(s_hbm, s_v)
        for c in range(n_chunks):
            off = base + c * chunk
            pltpu.sync_copy(idx_hbm.at[pl.ds(off, chunk)], idx_v)

            def body(j, _):
                sl = pl.ds(j * _L, _L)
                out_v[sl] = plsc.load_gather(s_v, [idx_v[sl]])
                return 0

            lax.fori_loop(0, chunk // _L, body, 0)
            pltpu.sync_copy(out_v, out_hbm.at[pl.ds(off, chunk)])

    return gather_kernel


def kernel(idseq, length_list, emb_table):
    b, sl = idseq.shape
    v, _ = emb_table.shape
    s = _rowsum_sigmoid(emb_table, blk=2000).reshape(v)
    gather = _make_sc_gather(v, b * sl, chunk=12800)
    out = gather(s, idseq.reshape(-1).astype(jnp.int32))
    return out.reshape(b, sl)


# speedup vs baseline: 32.8548x; 1.5520x over previous
import functools

import jax
import jax.numpy as jnp
from jax import lax
from jax.experimental import pallas as pl
from jax.experimental.pallas import tpu as pltpu
from jax.experimental.pallas import tpu_sc as plsc

_NC = 2
_NS = 16
_NW = _NC * _NS
_L = 16


def _rowsum_body(emb_ref, out_ref):
    x = emb_ref[...]
    ones = jnp.ones((1, x.shape[1]), jnp.float32)
    r = jax.lax.dot_general(
        ones, x, (((1,), (1,)), ((), ())),
        preferred_element_type=jnp.float32,
    )
    out_ref[...] = jax.nn.sigmoid(r).reshape(out_ref.shape)


def _rowsum_sigmoid(emb_table, blk, v_pad):
    v, d = emb_table.shape
    grid = v_pad // blk
    return pl.pallas_call(
        _rowsum_body,
        grid=(grid,),
        in_specs=[pl.BlockSpec((blk, d), lambda i: (i, 0))],
        out_specs=pl.BlockSpec((blk,), lambda i: (i,)),
        out_shape=jax.ShapeDtypeStruct((v_pad,), jnp.float32),
    )(emb_table)


def _make_sc_gather(v, total, chunk):
    per_w = total // _NW
    n_chunks = per_w // chunk
    mesh = plsc.VectorSubcoreMesh(core_axis_name="c", subcore_axis_name="s")

    @functools.partial(
        pl.kernel,
        mesh=mesh,
        out_type=jax.ShapeDtypeStruct((total,), jnp.float32),
        scratch_types=[
            pltpu.VMEM((v,), jnp.float32),
            pltpu.VMEM((chunk,), jnp.int32),
            pltpu.VMEM((chunk,), jnp.float32),
        ],
        compiler_params=pltpu.CompilerParams(needs_layout_passes=False),
    )
    def gather_kernel(s_hbm, idx_hbm, out_hbm, s_v, idx_v, out_v):
        wid = lax.axis_index("s") * _NC + lax.axis_index("c")
        base = wid * per_w
        pltpu.sync_copy(s_hbm, s_v)
        for c in range(n_chunks):
            off = base + c * chunk
            pltpu.sync_copy(idx_hbm.at[pl.ds(off, chunk)], idx_v)

            def body(j, _):
                sl = pl.ds(j * _L, _L)
                out_v[sl] = plsc.load_gather(s_v, [idx_v[sl]])
                return 0

            lax.fori_loop(0, chunk // _L, body, 0)
            pltpu.sync_copy(out_v, out_hbm.at[pl.ds(off, chunk)])

    return gather_kernel


def kernel(idseq, length_list, emb_table):
    b, sl = idseq.shape
    v, _ = emb_table.shape
    v_pad = 102400
    s = _rowsum_sigmoid(emb_table, blk=4096, v_pad=v_pad)
    gather = _make_sc_gather(v_pad, b * sl, chunk=12800)
    out = gather(s, idseq.reshape(-1).astype(jnp.int32))
    return out.reshape(b, sl)
